# baseline (speedup 1.0000x reference)
import functools

import jax
import jax.numpy as jnp
from jax import lax
from jax.experimental import pallas as pl
from jax.experimental.pallas import tpu as pltpu
from jax.experimental.pallas import tpu_sc as plsc

N = 10000
E = 640000
D_IN = 22
H = 128
C = 3
L = 3

RB = 1000
NB = N // RB
EB = 5000
NEB = E // EB

SC_NC = 2
SC_NS = 16
NW = SC_NC * SC_NS
CH = 80
NCH = E // (NW * CH)
NSB = 5
SB = NCH // NSB
ZS = 624

_SC_MESH = plsc.VectorSubcoreMesh(
    core_axis_name="c", subcore_axis_name="s",
    num_cores=SC_NC, num_subcores=SC_NS)


def _deg_body(dst_hbm, ew_hbm, out_hbm, dstv, eww, zb, degsh):
    cid = lax.axis_index("c")
    sid = lax.axis_index("s")
    wid = sid * SC_NC + cid

    def zb_body(i, _):
        zb[pl.ds(i * 16, 16)] = jnp.zeros((16,), jnp.float32)
        return 0
    lax.fori_loop(0, 40, zb_body, 0)

    @pl.when(sid < SC_NS - 1)
    def _():
        pltpu.sync_copy(zb.at[pl.ds(0, ZS)], degsh.at[pl.ds(sid * ZS, ZS)])

    @pl.when(sid == SC_NS - 1)
    def _():
        pltpu.sync_copy(zb, degsh.at[pl.ds((SC_NS - 1) * ZS, 640)])

    plsc.subcore_barrier()

    def sbody(sb, _):
        pltpu.sync_copy(dst_hbm.at[wid].at[sb], dstv)
        pltpu.sync_copy(ew_hbm.at[wid].at[sb], eww)

        def body(j, _):
            pltpu.sync_copy(eww.at[j], degsh.at[dstv.at[j]], add=True)
            return 0
        lax.fori_loop(0, SB, body, 0)
        return 0
    lax.fori_loop(0, NSB, sbody, 0)

    plsc.subcore_barrier()

    @pl.when(sid == 0)
    def _():
        pltpu.sync_copy(degsh, out_hbm.at[cid])


_sc_deg = pl.kernel(
    _deg_body,
    out_type=jax.ShapeDtypeStruct((SC_NC, N), jnp.float32),
    mesh=_SC_MESH,
    scratch_types=[
        pltpu.VMEM((SB, CH), jnp.int32),
        pltpu.VMEM((SB, CH), jnp.float32),
        pltpu.VMEM((640,), jnp.float32),
        pltpu.VMEM_SHARED((N,), jnp.float32),
    ],
)


def _scatter_body(hxp_hbm, src_hbm, dst_hbm, ew_hbm, out_hbm,
                  srcv, dstv, eww, rows0, rows1, accsh, semg0, semg1):
    cid = lax.axis_index("c")
    sid = lax.axis_index("s")
    wid = sid * SC_NC + cid

    def zr(i, _):
        for f in range(8):
            rows0[i, pl.ds(f * 16, 16)] = jnp.zeros((16,), jnp.float32)
        return 0
    lax.fori_loop(0, CH, zr, 0)

    @pl.when(sid < SC_NS - 1)
    def _():
        for k in range(ZS // CH):
            pltpu.sync_copy(rows0, accsh.at[pl.ds(sid * ZS + k * CH, CH)])
        pltpu.sync_copy(rows0.at[pl.ds(0, ZS % CH)],
                        accsh.at[pl.ds(sid * ZS + ZS - ZS % CH, ZS % CH)])

    @pl.when(sid == SC_NS - 1)
    def _():
        for k in range(8):
            pltpu.sync_copy(rows0, accsh.at[pl.ds((SC_NS - 1) * ZS + k * CH, CH)])

    plsc.subcore_barrier()

    def sbody(sb, _):
        pltpu.sync_copy(src_hbm.at[wid].at[sb], srcv)
        pltpu.sync_copy(dst_hbm.at[wid].at[sb], dstv)
        pltpu.sync_copy(ew_hbm.at[wid].at[sb], eww)
        pltpu.async_copy(hxp_hbm.at[srcv.at[0]], rows0, semg0)
        pltpu.async_copy(hxp_hbm.at[srcv.at[1]], rows1, semg1)

        def body(t, _):
            for k in range(2):
                rbuf = rows0 if k == 0 else rows1
                sg = semg0 if k == 0 else semg1
                j = 2 * t + k
                pltpu.make_async_copy(hxp_hbm.at[srcv.at[j]], rbuf, sg).wait()

                def eb(g, _):
                    w16 = eww[j, pl.ds(g * 16, 16)]
                    for ee in range(16):
                        wv = lax.broadcast(w16[ee], (16,))
                        e = g * 16 + ee
                        for f in range(8):
                            rbuf[e, pl.ds(f * 16, 16)] = \
                                rbuf[e, pl.ds(f * 16, 16)] * wv
                    return 0
                lax.fori_loop(0, CH // 16, eb, 0)
                pltpu.sync_copy(rbuf, accsh.at[dstv.at[j]], add=True)
                nj = j + 2

                @pl.when(nj < SB)
                def _():
                    pltpu.async_copy(hxp_hbm.at[srcv.at[nj]], rbuf, sg)
            return 0
        lax.fori_loop(0, SB // 2, body, 0)
        return 0
    lax.fori_loop(0, NSB, sbody, 0)

    plsc.subcore_barrier()

    @pl.when(sid < SC_NS - 1)
    def _():
        pltpu.sync_copy(accsh.at[pl.ds(sid * ZS, ZS)],
                        out_hbm.at[cid].at[pl.ds(sid * ZS, ZS)])

    @pl.when(sid == SC_NS - 1)
    def _():
        pltpu.sync_copy(accsh.at[pl.ds((SC_NS - 1) * ZS, 640)],
                        out_hbm.at[cid].at[pl.ds((SC_NS - 1) * ZS, 640)])


_sc_scatter = pl.kernel(
    _scatter_body,
    out_type=jax.ShapeDtypeStruct((SC_NC, N, H), jnp.float32),
    mesh=_SC_MESH,
    scratch_types=[
        pltpu.VMEM((SB, CH), jnp.int32),
        pltpu.VMEM((SB, CH), jnp.int32),
        pltpu.VMEM((SB, CH), jnp.float32),
        pltpu.VMEM((CH, H), jnp.float32),
        pltpu.VMEM((CH, H), jnp.float32),
        pltpu.VMEM_SHARED((N, H), jnp.float32),
        pltpu.SemaphoreType.DMA,
        pltpu.SemaphoreType.DMA,
    ],
)


KG = 5
NSB_G = 10
SB_G = 25
NCH_G = NSB_G * SB_G


def _gather2_body(a_hbm, b_hbm, src_hbm, dst_hbm, as_hbm, bd_hbm,
                  srcv, dstv,
                  ab0, ab1, ab2, ab3, ab4, bb0, bb1, bb2, bb3, bb4,
                  sa0, sa1, sa2, sa3, sa4, sb0, sb1, sb2, sb3, sb4):
    cid = lax.axis_index("c")
    sid = lax.axis_index("s")
    wid = sid * SC_NC + cid
    base = wid * NCH_G
    abufs = (ab0, ab1, ab2, ab3, ab4)
    bbufs = (bb0, bb1, bb2, bb3, bb4)
    sas = (sa0, sa1, sa2, sa3, sa4)
    sbs = (sb0, sb1, sb2, sb3, sb4)

    def sbody(sb, _):
        pltpu.sync_copy(src_hbm.at[wid].at[sb], srcv)
        pltpu.sync_copy(dst_hbm.at[wid].at[sb], dstv)
        for k in range(KG):
            pltpu.async_copy(a_hbm.at[srcv.at[k]], abufs[k], sas[k])
            pltpu.async_copy(b_hbm.at[dstv.at[k]], bbufs[k], sbs[k])

        def body(t, _):
            for k in range(KG):
                j = KG * t + k
                pltpu.make_async_copy(a_hbm.at[srcv.at[j]], abufs[k], sas[k]).wait()
                pltpu.make_async_copy(b_hbm.at[dstv.at[j]], bbufs[k], sbs[k]).wait()
                row0 = (base + sb * SB_G + j) * CH
                pltpu.sync_copy(abufs[k], as_hbm.at[pl.ds(row0, CH)])
                pltpu.sync_copy(bbufs[k], bd_hbm.at[pl.ds(row0, CH)])
                nj = j + KG

                @pl.when(nj < SB_G)
                def _():
                    pltpu.async_copy(a_hbm.at[srcv.at[nj]], abufs[k], sas[k])
                    pltpu.async_copy(b_hbm.at[dstv.at[nj]], bbufs[k], sbs[k])
            return 0
        lax.fori_loop(0, SB_G // KG, body, 0)
        return 0
    lax.fori_loop(0, NSB_G, sbody, 0)


_sc_gather2 = pl.kernel(
    _gather2_body,
    out_type=[
        jax.ShapeDtypeStruct((E, H // 2), jnp.int32),
        jax.ShapeDtypeStruct((E, H // 2), jnp.int32),
    ],
    mesh=_SC_MESH,
    scratch_types=(
        [pltpu.VMEM((SB_G, CH), jnp.int32)] * 2
        + [pltpu.VMEM((CH, H // 2), jnp.int32)] * (2 * KG)
        + [pltpu.SemaphoreType.DMA] * (2 * KG)
    ),
    compiler_params=pltpu.CompilerParams(use_tc_tiling_on_sc=False),
)


def _pre_body(x_ref, we_ref, be_ref, d0_ref, d1_ref, w0_ref, h_ref, hxp_ref, dinv_ref):
    deg = d0_ref[...] + d1_ref[...] + 1.0
    dinv = jnp.where(deg > 0, lax.rsqrt(deg), 0.0)
    h = jnp.maximum(jnp.dot(x_ref[...], we_ref[...],
                            preferred_element_type=jnp.float32) + be_ref[...], 0.0)
    h_ref[...] = h
    hxp_ref[...] = dinv * jnp.dot(h, w0_ref[...], preferred_element_type=jnp.float32)
    dinv_ref[...] = dinv


def _pre(x, W_enc, b_enc, d0, d1, W0):
    return pl.pallas_call(
        _pre_body,
        grid=(NB,),
        in_specs=[
            pl.BlockSpec((RB, D_IN), lambda i: (i, 0)),
            pl.BlockSpec((D_IN, H), lambda i: (0, 0)),
            pl.BlockSpec((1, H), lambda i: (0, 0)),
            pl.BlockSpec((RB, 1), lambda i: (i, 0)),
            pl.BlockSpec((RB, 1), lambda i: (i, 0)),
            pl.BlockSpec((H, H), lambda i: (0, 0)),
        ],
        out_specs=[
            pl.BlockSpec((RB, H), lambda i: (i, 0)),
            pl.BlockSpec((RB, H), lambda i: (i, 0)),
            pl.BlockSpec((RB, 1), lambda i: (i, 0)),
        ],
        out_shape=[
            jax.ShapeDtypeStruct((N, H), jnp.float32),
            jax.ShapeDtypeStruct((N, H), jnp.float32),
            jax.ShapeDtypeStruct((N, 1), jnp.float32),
        ],
    )(x, W_enc, b_enc, d0, d1, W0)


def _mid_body(h_ref, hxp_ref, ms0_ref, ms1_ref, dinv_ref, bi_ref, g_ref, bl_ref,
              wn_ref, hn_ref, hxn_ref):
    dinv = dinv_ref[...]
    agg = dinv * (ms0_ref[...] + ms1_ref[...] + hxp_ref[...]) + bi_ref[...]
    mu = jnp.mean(agg, axis=-1, keepdims=True)
    var = jnp.mean((agg - mu) ** 2, axis=-1, keepdims=True)
    u = (agg - mu) * lax.rsqrt(var + 1e-5) * g_ref[...] + bl_ref[...]
    hn = h_ref[...] + jnp.maximum(u, 0.0)
    hn_ref[...] = hn
    hxn_ref[...] = dinv * jnp.dot(hn, wn_ref[...], preferred_element_type=jnp.float32)


def _mid(h, hxp, ms0, ms1, dinv, bi, g, bl, Wn):
    return pl.pallas_call(
        _mid_body,
        grid=(NB,),
        in_specs=[
            pl.BlockSpec((RB, H), lambda i: (i, 0)),
            pl.BlockSpec((RB, H), lambda i: (i, 0)),
            pl.BlockSpec((RB, H), lambda i: (i, 0)),
            pl.BlockSpec((RB, H), lambda i: (i, 0)),
            pl.BlockSpec((RB, 1), lambda i: (i, 0)),
            pl.BlockSpec((1, H), lambda i: (0, 0)),
            pl.BlockSpec((1, H), lambda i: (0, 0)),
            pl.BlockSpec((1, H), lambda i: (0, 0)),
            pl.BlockSpec((H, H), lambda i: (0, 0)),
        ],
        out_specs=[
            pl.BlockSpec((RB, H), lambda i: (i, 0)),
            pl.BlockSpec((RB, H), lambda i: (i, 0)),
        ],
        out_shape=[
            jax.ShapeDtypeStruct((N, H), jnp.float32),
            jax.ShapeDtypeStruct((N, H), jnp.float32),
        ],
    )(h, hxp, ms0, ms1, dinv, bi, g, bl, Wn)


def _last_body(h_ref, hxp_ref, ms0_ref, ms1_ref, dinv_ref, bi_ref, g_ref, bl_ref,
               w1a_ref, w1b_ref, b1_ref, clw1_ref, clb1_ref, clw2_ref, clb2_ref,
               a_ref, b_ref, logits_ref, sum_acc, max_acc):
    step = pl.program_id(0)
    dinv = dinv_ref[...]
    agg = dinv * (ms0_ref[...] + ms1_ref[...] + hxp_ref[...]) + bi_ref[...]
    mu = jnp.mean(agg, axis=-1, keepdims=True)
    var = jnp.mean((agg - mu) ** 2, axis=-1, keepdims=True)
    u = (agg - mu) * lax.rsqrt(var + 1e-5) * g_ref[...] + bl_ref[...]
    hn = h_ref[...] + jnp.maximum(u, 0.0)
    a_ref[...] = (jnp.dot(hn, w1a_ref[...], preferred_element_type=jnp.float32)
                  + b1_ref[...]).astype(jnp.bfloat16)
    b_ref[...] = jnp.dot(hn, w1b_ref[...],
                         preferred_element_type=jnp.float32).astype(jnp.bfloat16)

    bsum = jnp.sum(hn, axis=0, keepdims=True)
    bmax = jnp.max(hn, axis=0, keepdims=True)

    @pl.when(step == 0)
    def _():
        sum_acc[...] = bsum
        max_acc[...] = bmax

    @pl.when(step > 0)
    def _():
        sum_acc[...] = sum_acc[...] + bsum
        max_acc[...] = jnp.maximum(max_acc[...], bmax)

    @pl.when(step == NB - 1)
    def _():
        hg = jnp.concatenate([sum_acc[...] * (1.0 / N), max_acc[...]], axis=1)
        z = jnp.maximum(jnp.dot(hg, clw1_ref[...],
                                preferred_element_type=jnp.float32) + clb1_ref[...], 0.0)
        logits_ref[...] = jnp.dot(z, clw2_ref[...],
                                  preferred_element_type=jnp.float32) + clb2_ref[...]


def _last(h, hxp, ms0, ms1, dinv, bi, g, bl, W1a, W1b, b1, clW1, clb1, clW2, clb2):
    return pl.pallas_call(
        _last_body,
        grid=(NB,),
        in_specs=[
            pl.BlockSpec((RB, H), lambda i: (i, 0)),
            pl.BlockSpec((RB, H), lambda i: (i, 0)),
            pl.BlockSpec((RB, H), lambda i: (i, 0)),
            pl.BlockSpec((RB, H), lambda i: (i, 0)),
            pl.BlockSpec((RB, 1), lambda i: (i, 0)),
            pl.BlockSpec((1, H), lambda i: (0, 0)),
            pl.BlockSpec((1, H), lambda i: (0, 0)),
            pl.BlockSpec((1, H), lambda i: (0, 0)),
            pl.BlockSpec((H, H), lambda i: (0, 0)),
            pl.BlockSpec((H, H), lambda i: (0, 0)),
            pl.BlockSpec((1, H), lambda i: (0, 0)),
            pl.BlockSpec((2 * H, H), lambda i: (0, 0)),
            pl.BlockSpec((1, H), lambda i: (0, 0)),
            pl.BlockSpec((H, C), lambda i: (0, 0)),
            pl.BlockSpec((1, C), lambda i: (0, 0)),
        ],
        out_specs=[
            pl.BlockSpec((RB, H), lambda i: (i, 0)),
            pl.BlockSpec((RB, H), lambda i: (i, 0)),
            pl.BlockSpec((1, C), lambda i: (0, 0)),
        ],
        out_shape=[
            jax.ShapeDtypeStruct((N, H), jnp.bfloat16),
            jax.ShapeDtypeStruct((N, H), jnp.bfloat16),
            jax.ShapeDtypeStruct((1, C), jnp.float32),
        ],
        scratch_shapes=[
            pltpu.VMEM((1, H), jnp.float32),
            pltpu.VMEM((1, H), jnp.float32),
        ],
    )(h, hxp, ms0, ms1, dinv, bi, g, bl, W1a, W1b, b1, clW1, clb1, clW2, clb2)


def _unpack_bf16_pair(x):
    lo = lax.bitcast_convert_type(lax.shift_left(x, 16), jnp.float32)
    hi = lax.bitcast_convert_type(
        lax.bitwise_and(x, jnp.int32(-65536)), jnp.float32)
    return lo, hi


def _ep_body(a_ref, b_ref, ew_ref, ce_ref, co_ref, w2e_ref, w2o_ref,
             b2_ref, w3_ref, b3_ref, s_ref):
    ae, ao = _unpack_bf16_pair(a_ref[...])
    be, bo = _unpack_bf16_pair(b_ref[...])
    ew = ew_ref[...]
    e1e = jnp.maximum(ae + be + ew * ce_ref[...], 0.0)
    e1o = jnp.maximum(ao + bo + ew * co_ref[...], 0.0)
    e2 = jnp.maximum(
        jnp.dot(e1e, w2e_ref[...], preferred_element_type=jnp.float32)
        + jnp.dot(e1o, w2o_ref[...], preferred_element_type=jnp.float32)
        + b2_ref[...], 0.0)
    z = jnp.dot(e2, w3_ref[...], preferred_element_type=jnp.float32) + b3_ref[...]
    s_ref[...] = 1.0 / (1.0 + jnp.exp(-z))


def _ep_tail(asrc, bdst, ew, ce, co, W2e, W2o, b2, W3, b3):
    ne = asrc.shape[0]
    return pl.pallas_call(
        _ep_body,
        grid=(ne // EB,),
        in_specs=[
            pl.BlockSpec((EB, H // 2), lambda i: (i, 0)),
            pl.BlockSpec((EB, H // 2), lambda i: (i, 0)),
            pl.BlockSpec((EB, 1), lambda i: (i, 0)),
            pl.BlockSpec((1, H // 2), lambda i: (0, 0)),
            pl.BlockSpec((1, H // 2), lambda i: (0, 0)),
            pl.BlockSpec((H // 2, 32), lambda i: (0, 0)),
            pl.BlockSpec((H // 2, 32), lambda i: (0, 0)),
            pl.BlockSpec((1, 32), lambda i: (0, 0)),
            pl.BlockSpec((32, 1), lambda i: (0, 0)),
            pl.BlockSpec((1, 1), lambda i: (0, 0)),
        ],
        out_specs=pl.BlockSpec((EB, 1), lambda i: (i, 0)),
        out_shape=jax.ShapeDtypeStruct((ne, 1), jnp.float32),
    )(asrc, bdst, ew, ce, co, W2e, W2o, b2, W3, b3)


def kernel(x, edge_index, edge_attr, W_enc, b_enc, conv_W, conv_b, ln_g, ln_b,
           ep_W1, ep_b1, ep_W2, ep_b2, ep_W3, ep_b3, cl_W1, cl_b1, cl_W2, cl_b2):
    src2d = edge_index[0].reshape(NW, NSB, SB, CH)
    dst2d = edge_index[1].reshape(NW, NSB, SB, CH)
    ew2d = edge_attr[:, 0].reshape(NW, NSB, SB, CH)

    degp = _sc_deg(dst2d, ew2d)
    h, hxp, dinv = _pre(x, W_enc, b_enc.reshape(1, H),
                        degp[0].reshape(N, 1), degp[1].reshape(N, 1), conv_W[0])

    for i in range(L):
        parts = _sc_scatter(hxp, src2d, dst2d, ew2d)
        if i < L - 1:
            h, hxp = _mid(h, hxp, parts[0], parts[1], dinv, conv_b[i].reshape(1, H),
                          ln_g[i].reshape(1, H), ln_b[i].reshape(1, H), conv_W[i + 1])
        else:
            A, B, logits = _last(
                h, hxp, parts[0], parts[1], dinv, conv_b[i].reshape(1, H),
                ln_g[i].reshape(1, H), ln_b[i].reshape(1, H),
                ep_W1[:H], ep_W1[H:2 * H], ep_b1.reshape(1, H),
                cl_W1, cl_b1.reshape(1, H), cl_W2, cl_b2.reshape(1, C))

    srcg = edge_index[0].reshape(NW, NSB_G, SB_G, CH)
    dstg = edge_index[1].reshape(NW, NSB_G, SB_G, CH)
    a32 = lax.bitcast_convert_type(A.reshape(N, H // 2, 2), jnp.int32)
    b32 = lax.bitcast_convert_type(B.reshape(N, H // 2, 2), jnp.int32)
    asrc, bdst = _sc_gather2(a32, b32, srcg, dstg)
    c_row = ep_W1[2 * H]
    s = _ep_tail(asrc, bdst, edge_attr,
                 c_row[0::2].reshape(1, H // 2), c_row[1::2].reshape(1, H // 2),
                 ep_W2[0::2], ep_W2[1::2],
                 ep_b2.reshape(1, 32), ep_W3, ep_b3.reshape(1, 1))
    return (logits, s[:, 0])


# speedup vs baseline: 1.1631x; 1.1631x over previous
import functools

import jax
import jax.numpy as jnp
from jax import lax
from jax.experimental import pallas as pl
from jax.experimental.pallas import tpu as pltpu
from jax.experimental.pallas import tpu_sc as plsc

N = 10000
E = 640000
D_IN = 22
H = 128
C = 3
L = 3

RB = 1000
NB = N // RB
EB = 5000
NEB = E // EB

SC_NC = 2
SC_NS = 16
NW = SC_NC * SC_NS
CH = 80
NCH = E // (NW * CH)
NSB = 5
SB = NCH // NSB
ZS = 624

_SC_MESH = plsc.VectorSubcoreMesh(
    core_axis_name="c", subcore_axis_name="s",
    num_cores=SC_NC, num_subcores=SC_NS)


def _deg_body(dst_hbm, ew_hbm, out_hbm, dstv, eww, zb, degsh):
    cid = lax.axis_index("c")
    sid = lax.axis_index("s")
    wid = sid * SC_NC + cid

    def zb_body(i, _):
        zb[pl.ds(i * 16, 16)] = jnp.zeros((16,), jnp.float32)
        return 0
    lax.fori_loop(0, 40, zb_body, 0)

    @pl.when(sid < SC_NS - 1)
    def _():
        pltpu.sync_copy(zb.at[pl.ds(0, ZS)], degsh.at[pl.ds(sid * ZS, ZS)])

    @pl.when(sid == SC_NS - 1)
    def _():
        pltpu.sync_copy(zb, degsh.at[pl.ds((SC_NS - 1) * ZS, 640)])

    plsc.subcore_barrier()

    def sbody(sb, _):
        pltpu.sync_copy(dst_hbm.at[wid].at[sb], dstv)
        pltpu.sync_copy(ew_hbm.at[wid].at[sb], eww)

        def body(j, _):
            pltpu.sync_copy(eww.at[j], degsh.at[dstv.at[j]], add=True)
            return 0
        lax.fori_loop(0, SB, body, 0)
        return 0
    lax.fori_loop(0, NSB, sbody, 0)

    plsc.subcore_barrier()

    @pl.when(sid == 0)
    def _():
        pltpu.sync_copy(degsh, out_hbm.at[cid])


_sc_deg = pl.kernel(
    _deg_body,
    out_type=jax.ShapeDtypeStruct((SC_NC, N), jnp.float32),
    mesh=_SC_MESH,
    scratch_types=[
        pltpu.VMEM((SB, CH), jnp.int32),
        pltpu.VMEM((SB, CH), jnp.float32),
        pltpu.VMEM((640,), jnp.float32),
        pltpu.VMEM_SHARED((N,), jnp.float32),
    ],
)


def _scatter_body(hxp_hbm, src_hbm, dst_hbm, ew_hbm, out_hbm,
                  srcv, dstv, eww, rows0, rows1, accsh, semg0, semg1):
    cid = lax.axis_index("c")
    sid = lax.axis_index("s")
    wid = sid * SC_NC + cid

    def zr(i, _):
        for f in range(8):
            rows0[i, pl.ds(f * 16, 16)] = jnp.zeros((16,), jnp.float32)
        return 0
    lax.fori_loop(0, CH, zr, 0)

    @pl.when(sid < SC_NS - 1)
    def _():
        for k in range(ZS // CH):
            pltpu.sync_copy(rows0, accsh.at[pl.ds(sid * ZS + k * CH, CH)])
        pltpu.sync_copy(rows0.at[pl.ds(0, ZS % CH)],
                        accsh.at[pl.ds(sid * ZS + ZS - ZS % CH, ZS % CH)])

    @pl.when(sid == SC_NS - 1)
    def _():
        for k in range(8):
            pltpu.sync_copy(rows0, accsh.at[pl.ds((SC_NS - 1) * ZS + k * CH, CH)])

    plsc.subcore_barrier()

    def sbody(sb, _):
        pltpu.sync_copy(src_hbm.at[wid].at[sb], srcv)
        pltpu.sync_copy(dst_hbm.at[wid].at[sb], dstv)
        pltpu.sync_copy(ew_hbm.at[wid].at[sb], eww)
        pltpu.async_copy(hxp_hbm.at[srcv.at[0]], rows0, semg0)
        pltpu.async_copy(hxp_hbm.at[srcv.at[1]], rows1, semg1)

        def body(t, _):
            for k in range(2):
                rbuf = rows0 if k == 0 else rows1
                sg = semg0 if k == 0 else semg1
                j = 2 * t + k
                pltpu.make_async_copy(hxp_hbm.at[srcv.at[j]], rbuf, sg).wait()

                def eb(g, _):
                    w16 = eww[j, pl.ds(g * 16, 16)]
                    for ee in range(16):
                        wv = lax.broadcast(w16[ee], (16,))
                        e = g * 16 + ee
                        for f in range(8):
                            rbuf[e, pl.ds(f * 16, 16)] = \
                                rbuf[e, pl.ds(f * 16, 16)] * wv
                    return 0
                lax.fori_loop(0, CH // 16, eb, 0)
                pltpu.sync_copy(rbuf, accsh.at[dstv.at[j]], add=True)
                nj = j + 2

                @pl.when(nj < SB)
                def _():
                    pltpu.async_copy(hxp_hbm.at[srcv.at[nj]], rbuf, sg)
            return 0
        lax.fori_loop(0, SB // 2, body, 0)
        return 0
    lax.fori_loop(0, NSB, sbody, 0)

    plsc.subcore_barrier()

    @pl.when(sid < SC_NS - 1)
    def _():
        pltpu.sync_copy(accsh.at[pl.ds(sid * ZS, ZS)],
                        out_hbm.at[cid].at[pl.ds(sid * ZS, ZS)])

    @pl.when(sid == SC_NS - 1)
    def _():
        pltpu.sync_copy(accsh.at[pl.ds((SC_NS - 1) * ZS, 640)],
                        out_hbm.at[cid].at[pl.ds((SC_NS - 1) * ZS, 640)])


_sc_scatter = pl.kernel(
    _scatter_body,
    out_type=jax.ShapeDtypeStruct((SC_NC, N, H), jnp.float32),
    mesh=_SC_MESH,
    scratch_types=[
        pltpu.VMEM((SB, CH), jnp.int32),
        pltpu.VMEM((SB, CH), jnp.int32),
        pltpu.VMEM((SB, CH), jnp.float32),
        pltpu.VMEM((CH, H), jnp.float32),
        pltpu.VMEM((CH, H), jnp.float32),
        pltpu.VMEM_SHARED((N, H), jnp.float32),
        pltpu.SemaphoreType.DMA,
        pltpu.SemaphoreType.DMA,
    ],
)


KG = 5
NSB_G = 10
SB_G = 25
NCH_G = NSB_G * SB_G


def _gather2_body(a_hbm, b_hbm, src_hbm, dst_hbm, as_hbm, bd_hbm,
                  srcv, dstv,
                  ab0, ab1, ab2, ab3, ab4, bb0, bb1, bb2, bb3, bb4,
                  sa0, sa1, sa2, sa3, sa4, sb0, sb1, sb2, sb3, sb4):
    cid = lax.axis_index("c")
    sid = lax.axis_index("s")
    wid = sid * SC_NC + cid
    base = wid * NCH_G
    abufs = (ab0, ab1, ab2, ab3, ab4)
    bbufs = (bb0, bb1, bb2, bb3, bb4)
    sas = (sa0, sa1, sa2, sa3, sa4)
    sbs = (sb0, sb1, sb2, sb3, sb4)

    def sbody(sb, _):
        pltpu.sync_copy(src_hbm.at[wid].at[sb], srcv)
        pltpu.sync_copy(dst_hbm.at[wid].at[sb], dstv)
        for k in range(KG):
            pltpu.async_copy(a_hbm.at[srcv.at[k]], abufs[k], sas[k])
            pltpu.async_copy(b_hbm.at[dstv.at[k]], bbufs[k], sbs[k])

        def body(t, _):
            for k in range(KG):
                j = KG * t + k
                pltpu.make_async_copy(a_hbm.at[srcv.at[j]], abufs[k], sas[k]).wait()
                pltpu.make_async_copy(b_hbm.at[dstv.at[j]], bbufs[k], sbs[k]).wait()
                row0 = (base + sb * SB_G + j) * CH
                pltpu.sync_copy(abufs[k], as_hbm.at[pl.ds(row0, CH)])
                pltpu.sync_copy(bbufs[k], bd_hbm.at[pl.ds(row0, CH)])
                nj = j + KG

                @pl.when(nj < SB_G)
                def _():
                    pltpu.async_copy(a_hbm.at[srcv.at[nj]], abufs[k], sas[k])
                    pltpu.async_copy(b_hbm.at[dstv.at[nj]], bbufs[k], sbs[k])
            return 0
        lax.fori_loop(0, SB_G // KG, body, 0)
        return 0
    lax.fori_loop(0, NSB_G, sbody, 0)


_sc_gather2 = pl.kernel(
    _gather2_body,
    out_type=[
        jax.ShapeDtypeStruct((E, H), jnp.float32),
        jax.ShapeDtypeStruct((E, H), jnp.float32),
    ],
    mesh=_SC_MESH,
    scratch_types=(
        [pltpu.VMEM((SB_G, CH), jnp.int32)] * 2
        + [pltpu.VMEM((CH, H), jnp.float32)] * (2 * KG)
        + [pltpu.SemaphoreType.DMA] * (2 * KG)
    ),
)


def _pre_body(x_ref, we_ref, be_ref, d0_ref, d1_ref, w0_ref, h_ref, hxp_ref, dinv_ref):
    deg = d0_ref[...] + d1_ref[...] + 1.0
    dinv = jnp.where(deg > 0, lax.rsqrt(deg), 0.0)
    h = jnp.maximum(jnp.dot(x_ref[...], we_ref[...],
                            preferred_element_type=jnp.float32) + be_ref[...], 0.0)
    h_ref[...] = h
    hxp_ref[...] = dinv * jnp.dot(h, w0_ref[...], preferred_element_type=jnp.float32)
    dinv_ref[...] = dinv


def _pre(x, W_enc, b_enc, d0, d1, W0):
    return pl.pallas_call(
        _pre_body,
        grid=(NB,),
        in_specs=[
            pl.BlockSpec((RB, D_IN), lambda i: (i, 0)),
            pl.BlockSpec((D_IN, H), lambda i: (0, 0)),
            pl.BlockSpec((1, H), lambda i: (0, 0)),
            pl.BlockSpec((RB, 1), lambda i: (i, 0)),
            pl.BlockSpec((RB, 1), lambda i: (i, 0)),
            pl.BlockSpec((H, H), lambda i: (0, 0)),
        ],
        out_specs=[
            pl.BlockSpec((RB, H), lambda i: (i, 0)),
            pl.BlockSpec((RB, H), lambda i: (i, 0)),
            pl.BlockSpec((RB, 1), lambda i: (i, 0)),
        ],
        out_shape=[
            jax.ShapeDtypeStruct((N, H), jnp.float32),
            jax.ShapeDtypeStruct((N, H), jnp.float32),
            jax.ShapeDtypeStruct((N, 1), jnp.float32),
        ],
    )(x, W_enc, b_enc, d0, d1, W0)


def _mid_body(h_ref, hxp_ref, ms0_ref, ms1_ref, dinv_ref, bi_ref, g_ref, bl_ref,
              wn_ref, hn_ref, hxn_ref):
    dinv = dinv_ref[...]
    agg = dinv * (ms0_ref[...] + ms1_ref[...] + hxp_ref[...]) + bi_ref[...]
    mu = jnp.mean(agg, axis=-1, keepdims=True)
    var = jnp.mean((agg - mu) ** 2, axis=-1, keepdims=True)
    u = (agg - mu) * lax.rsqrt(var + 1e-5) * g_ref[...] + bl_ref[...]
    hn = h_ref[...] + jnp.maximum(u, 0.0)
    hn_ref[...] = hn
    hxn_ref[...] = dinv * jnp.dot(hn, wn_ref[...], preferred_element_type=jnp.float32)


def _mid(h, hxp, ms0, ms1, dinv, bi, g, bl, Wn):
    return pl.pallas_call(
        _mid_body,
        grid=(NB,),
        in_specs=[
            pl.BlockSpec((RB, H), lambda i: (i, 0)),
            pl.BlockSpec((RB, H), lambda i: (i, 0)),
            pl.BlockSpec((RB, H), lambda i: (i, 0)),
            pl.BlockSpec((RB, H), lambda i: (i, 0)),
            pl.BlockSpec((RB, 1), lambda i: (i, 0)),
            pl.BlockSpec((1, H), lambda i: (0, 0)),
            pl.BlockSpec((1, H), lambda i: (0, 0)),
            pl.BlockSpec((1, H), lambda i: (0, 0)),
            pl.BlockSpec((H, H), lambda i: (0, 0)),
        ],
        out_specs=[
            pl.BlockSpec((RB, H), lambda i: (i, 0)),
            pl.BlockSpec((RB, H), lambda i: (i, 0)),
        ],
        out_shape=[
            jax.ShapeDtypeStruct((N, H), jnp.float32),
            jax.ShapeDtypeStruct((N, H), jnp.float32),
        ],
    )(h, hxp, ms0, ms1, dinv, bi, g, bl, Wn)


def _last_body(h_ref, hxp_ref, ms0_ref, ms1_ref, dinv_ref, bi_ref, g_ref, bl_ref,
               w1a_ref, w1b_ref, b1_ref, clw1_ref, clb1_ref, clw2_ref, clb2_ref,
               a_ref, b_ref, logits_ref, sum_acc, max_acc):
    step = pl.program_id(0)
    dinv = dinv_ref[...]
    agg = dinv * (ms0_ref[...] + ms1_ref[...] + hxp_ref[...]) + bi_ref[...]
    mu = jnp.mean(agg, axis=-1, keepdims=True)
    var = jnp.mean((agg - mu) ** 2, axis=-1, keepdims=True)
    u = (agg - mu) * lax.rsqrt(var + 1e-5) * g_ref[...] + bl_ref[...]
    hn = h_ref[...] + jnp.maximum(u, 0.0)
    a_ref[...] = jnp.dot(hn, w1a_ref[...], preferred_element_type=jnp.float32) + b1_ref[...]
    b_ref[...] = jnp.dot(hn, w1b_ref[...], preferred_element_type=jnp.float32)

    bsum = jnp.sum(hn, axis=0, keepdims=True)
    bmax = jnp.max(hn, axis=0, keepdims=True)

    @pl.when(step == 0)
    def _():
        sum_acc[...] = bsum
        max_acc[...] = bmax

    @pl.when(step > 0)
    def _():
        sum_acc[...] = sum_acc[...] + bsum
        max_acc[...] = jnp.maximum(max_acc[...], bmax)

    @pl.when(step == NB - 1)
    def _():
        hg = jnp.concatenate([sum_acc[...] * (1.0 / N), max_acc[...]], axis=1)
        z = jnp.maximum(jnp.dot(hg, clw1_ref[...],
                                preferred_element_type=jnp.float32) + clb1_ref[...], 0.0)
        logits_ref[...] = jnp.dot(z, clw2_ref[...],
                                  preferred_element_type=jnp.float32) + clb2_ref[...]


def _last(h, hxp, ms0, ms1, dinv, bi, g, bl, W1a, W1b, b1, clW1, clb1, clW2, clb2):
    return pl.pallas_call(
        _last_body,
        grid=(NB,),
        in_specs=[
            pl.BlockSpec((RB, H), lambda i: (i, 0)),
            pl.BlockSpec((RB, H), lambda i: (i, 0)),
            pl.BlockSpec((RB, H), lambda i: (i, 0)),
            pl.BlockSpec((RB, H), lambda i: (i, 0)),
            pl.BlockSpec((RB, 1), lambda i: (i, 0)),
            pl.BlockSpec((1, H), lambda i: (0, 0)),
            pl.BlockSpec((1, H), lambda i: (0, 0)),
            pl.BlockSpec((1, H), lambda i: (0, 0)),
            pl.BlockSpec((H, H), lambda i: (0, 0)),
            pl.BlockSpec((H, H), lambda i: (0, 0)),
            pl.BlockSpec((1, H), lambda i: (0, 0)),
            pl.BlockSpec((2 * H, H), lambda i: (0, 0)),
            pl.BlockSpec((1, H), lambda i: (0, 0)),
            pl.BlockSpec((H, C), lambda i: (0, 0)),
            pl.BlockSpec((1, C), lambda i: (0, 0)),
        ],
        out_specs=[
            pl.BlockSpec((RB, H), lambda i: (i, 0)),
            pl.BlockSpec((RB, H), lambda i: (i, 0)),
            pl.BlockSpec((1, C), lambda i: (0, 0)),
        ],
        out_shape=[
            jax.ShapeDtypeStruct((N, H), jnp.float32),
            jax.ShapeDtypeStruct((N, H), jnp.float32),
            jax.ShapeDtypeStruct((1, C), jnp.float32),
        ],
        scratch_shapes=[
            pltpu.VMEM((1, H), jnp.float32),
            pltpu.VMEM((1, H), jnp.float32),
        ],
    )(h, hxp, ms0, ms1, dinv, bi, g, bl, W1a, W1b, b1, clW1, clb1, clW2, clb2)


def _ep_body(a_ref, b_ref, ew_ref, c_ref, w2_ref, b2_ref, w3_ref, b3_ref, s_ref):
    e1 = jnp.maximum(a_ref[...] + b_ref[...] + ew_ref[...] * c_ref[...], 0.0)
    e2 = jnp.maximum(jnp.dot(e1, w2_ref[...],
                             preferred_element_type=jnp.float32) + b2_ref[...], 0.0)
    z = jnp.dot(e2, w3_ref[...], preferred_element_type=jnp.float32) + b3_ref[...]
    s_ref[...] = 1.0 / (1.0 + jnp.exp(-z))


def _ep_tail(asrc, bdst, ew, c_row, W2, b2, W3, b3):
    ne = asrc.shape[0]
    return pl.pallas_call(
        _ep_body,
        grid=(ne // EB,),
        in_specs=[
            pl.BlockSpec((EB, H), lambda i: (i, 0)),
            pl.BlockSpec((EB, H), lambda i: (i, 0)),
            pl.BlockSpec((EB, 1), lambda i: (i, 0)),
            pl.BlockSpec((1, H), lambda i: (0, 0)),
            pl.BlockSpec((H, 32), lambda i: (0, 0)),
            pl.BlockSpec((1, 32), lambda i: (0, 0)),
            pl.BlockSpec((32, 1), lambda i: (0, 0)),
            pl.BlockSpec((1, 1), lambda i: (0, 0)),
        ],
        out_specs=pl.BlockSpec((EB, 1), lambda i: (i, 0)),
        out_shape=jax.ShapeDtypeStruct((ne, 1), jnp.float32),
    )(asrc, bdst, ew, c_row, W2, b2, W3, b3)


def kernel(x, edge_index, edge_attr, W_enc, b_enc, conv_W, conv_b, ln_g, ln_b,
           ep_W1, ep_b1, ep_W2, ep_b2, ep_W3, ep_b3, cl_W1, cl_b1, cl_W2, cl_b2):
    src2d = edge_index[0].reshape(NW, NSB, SB, CH)
    dst2d = edge_index[1].reshape(NW, NSB, SB, CH)
    ew2d = edge_attr[:, 0].reshape(NW, NSB, SB, CH)

    degp = _sc_deg(dst2d, ew2d)
    h, hxp, dinv = _pre(x, W_enc, b_enc.reshape(1, H),
                        degp[0].reshape(N, 1), degp[1].reshape(N, 1), conv_W[0])

    for i in range(L):
        parts = _sc_scatter(hxp, src2d, dst2d, ew2d)
        if i < L - 1:
            h, hxp = _mid(h, hxp, parts[0], parts[1], dinv, conv_b[i].reshape(1, H),
                          ln_g[i].reshape(1, H), ln_b[i].reshape(1, H), conv_W[i + 1])
        else:
            A, B, logits = _last(
                h, hxp, parts[0], parts[1], dinv, conv_b[i].reshape(1, H),
                ln_g[i].reshape(1, H), ln_b[i].reshape(1, H),
                ep_W1[:H], ep_W1[H:2 * H], ep_b1.reshape(1, H),
                cl_W1, cl_b1.reshape(1, H), cl_W2, cl_b2.reshape(1, C))

    srcg = edge_index[0].reshape(NW, NSB_G, SB_G, CH)
    dstg = edge_index[1].reshape(NW, NSB_G, SB_G, CH)
    asrc, bdst = _sc_gather2(A, B, srcg, dstg)
    s = _ep_tail(asrc, bdst, edge_attr, ep_W1[2 * H].reshape(1, H),
                 ep_W2, ep_b2.reshape(1, 32), ep_W3, ep_b3.reshape(1, 1))
    return (logits, s[:, 0])
